# baseline (speedup 1.0000x reference)
import jax
import jax.numpy as jnp
from jax.experimental import pallas as pl

_K = 1024
_D = 64
_Q = 2


def _rvq_kernel(x_ref, cb_ref, out_ref, idx0_ref, idx1_ref, loss_ref):
    x = x_ref[0]
    residual = x
    quant_out = jnp.zeros_like(x)
    losses = []
    idx_refs = (idx0_ref, idx1_ref)
    for q in range(_Q):
        cb = cb_ref[q]
        cnorm = jnp.sum(cb * cb, axis=1)
        rnorm = jnp.sum(residual * residual, axis=1)
        prod = jax.lax.dot_general(
            residual, cb, (((1,), (1,)), ((), ())),
            preferred_element_type=jnp.float32,
            precision=jax.lax.Precision.DEFAULT)
        d = (rnorm[:, None] - 2.0 * prod) + cnorm[None, :]
        dmin = jnp.min(d, axis=1)
        iota = jax.lax.broadcasted_iota(jnp.int32, d.shape, 1)
        idx = jnp.min(jnp.where(d == dmin[:, None], iota, _K), axis=1)
        idx_refs[q][0, 0] = idx
        onehot = (iota == idx[:, None]).astype(jnp.float32)
        quant = jax.lax.dot_general(
            onehot, cb, (((1,), (0,)), ((), ())),
            preferred_element_type=jnp.float32,
            precision=jax.lax.Precision.HIGHEST)
        diff = quant - residual
        losses.append(jnp.sum(diff * diff))
        quant_out = quant_out + (residual + (quant - residual))
        residual = residual - quant
    out_ref[0] = quant_out
    loss_ref[0] = jnp.stack([jnp.broadcast_to(l, (128,)) for l in losses])


def _rvq(x, codebooks, tb, interpret=False):
    b, t, d_ = x.shape
    g = t // tb
    out, idx0, idx1, lossp = pl.pallas_call(
        _rvq_kernel,
        grid=(b, g),
        in_specs=[
            pl.BlockSpec((1, tb, d_), lambda i, j: (i, j, 0)),
            pl.BlockSpec((_Q, _K, d_), lambda i, j: (0, 0, 0)),
        ],
        out_specs=[
            pl.BlockSpec((1, tb, d_), lambda i, j: (i, j, 0)),
            pl.BlockSpec((1, 1, tb), lambda i, j: (i * g + j, 0, 0)),
            pl.BlockSpec((1, 1, tb), lambda i, j: (i * g + j, 0, 0)),
            pl.BlockSpec((1, _Q, 128), lambda i, j: (i * g + j, 0, 0)),
        ],
        out_shape=[
            jax.ShapeDtypeStruct((b, t, d_), jnp.float32),
            jax.ShapeDtypeStruct((b * g, 1, tb), jnp.int32),
            jax.ShapeDtypeStruct((b * g, 1, tb), jnp.int32),
            jax.ShapeDtypeStruct((b * g, _Q, 128), jnp.float32),
        ],
        interpret=interpret,
    )(x, codebooks)
    indices = jnp.stack([idx0.reshape(b, t), idx1.reshape(b, t)], axis=-1)
    loss = lossp[:, :, 0].sum(axis=0) / (b * t * d_)
    return out, indices, loss


def kernel(audio_input, image_input, audio_codebooks, image_codebooks):
    xa = jnp.transpose(audio_input, (0, 2, 1))
    qa, audio_indices, vq_audio_loss = _rvq(xa, audio_codebooks, 512)
    recon_audio = jnp.transpose(qa, (0, 2, 1))

    bi, di, h, w = image_input.shape
    xi = jnp.transpose(image_input, (0, 2, 3, 1)).reshape(bi, h * w, di)
    qi, image_indices, vq_image_loss = _rvq(xi, image_codebooks, 512)
    recon_image = jnp.transpose(qi.reshape(bi, h, w, di), (0, 3, 1, 2))

    return (recon_audio, recon_image, vq_audio_loss, vq_image_loss,
            audio_indices, image_indices)


# speedup vs baseline: 1.8255x; 1.8255x over previous
import jax
import jax.numpy as jnp
from jax.experimental import pallas as pl

_K = 1024
_D = 64
_Q = 2


def _dot(a, b, dims):
    return jax.lax.dot_general(a, b, (dims, ((), ())),
                               preferred_element_type=jnp.float32,
                               precision=jax.lax.Precision.DEFAULT)


def _rvq_kernel(x_ref, cb_ref, sp_ref, out_ref, idx0_ref, idx1_ref, loss_ref):
    x = x_ref[0]
    residual = x
    quant_out = jnp.zeros_like(x)
    losses = []
    idx_refs = (idx0_ref, idx1_ref)
    for q in range(_Q):
        cb = cb_ref[q]
        cnorm = jnp.sum(cb * cb, axis=1)
        rnorm = jnp.sum(residual * residual, axis=1)
        prod = _dot(residual, cb, ((1,), (1,)))
        d = (rnorm[:, None] - 2.0 * prod) + cnorm[None, :]
        dmin = jnp.min(d, axis=1)
        iota = jax.lax.broadcasted_iota(jnp.int32, d.shape, 1)
        idx = jnp.min(jnp.where(d == dmin[:, None], iota, _K), axis=1)
        idx_refs[q][0, 0] = idx
        onehot = (iota == idx[:, None]).astype(jnp.float32)
        if q == 0:
            quant = ((_dot(onehot, sp_ref[0], ((1,), (0,)))
                      + _dot(onehot, sp_ref[1], ((1,), (0,))))
                     + _dot(onehot, sp_ref[2], ((1,), (0,))))
        else:
            quant = _dot(onehot, cb, ((1,), (0,)))
        diff = quant - residual
        losses.append(jnp.sum(diff * diff))
        quant_out = quant_out + (residual + (quant - residual))
        residual = residual - quant
    out_ref[0] = quant_out
    loss_ref[0] = jnp.stack([jnp.broadcast_to(l, (128,)) for l in losses])


def _split3(cb):
    mask = jnp.int32(-65536)
    hi = jnp.bitwise_and(cb.view(jnp.int32), mask).view(jnp.float32)
    r = cb - hi
    mid = jnp.bitwise_and(r.view(jnp.int32), mask).view(jnp.float32)
    lo = r - mid
    return jnp.stack([hi, mid, lo])


def _rvq(x, codebooks, tb, interpret=False):
    b, t, d_ = x.shape
    g = t // tb
    out, idx0, idx1, lossp = pl.pallas_call(
        _rvq_kernel,
        grid=(b, g),
        in_specs=[
            pl.BlockSpec((1, tb, d_), lambda i, j: (i, j, 0)),
            pl.BlockSpec((_Q, _K, d_), lambda i, j: (0, 0, 0)),
            pl.BlockSpec((3, _K, d_), lambda i, j: (0, 0, 0)),
        ],
        out_specs=[
            pl.BlockSpec((1, tb, d_), lambda i, j: (i, j, 0)),
            pl.BlockSpec((1, 1, tb), lambda i, j: (i * g + j, 0, 0)),
            pl.BlockSpec((1, 1, tb), lambda i, j: (i * g + j, 0, 0)),
            pl.BlockSpec((1, _Q, 128), lambda i, j: (i * g + j, 0, 0)),
        ],
        out_shape=[
            jax.ShapeDtypeStruct((b, t, d_), jnp.float32),
            jax.ShapeDtypeStruct((b * g, 1, tb), jnp.int32),
            jax.ShapeDtypeStruct((b * g, 1, tb), jnp.int32),
            jax.ShapeDtypeStruct((b * g, _Q, 128), jnp.float32),
        ],
        interpret=interpret,
    )(x, codebooks, _split3(codebooks[0]))
    indices = jnp.stack([idx0.reshape(b, t), idx1.reshape(b, t)], axis=-1)
    loss = lossp[:, :, 0].sum(axis=0) / (b * t * d_)
    return out, indices, loss


def kernel(audio_input, image_input, audio_codebooks, image_codebooks):
    xa = jnp.transpose(audio_input, (0, 2, 1))
    qa, audio_indices, vq_audio_loss = _rvq(xa, audio_codebooks, 512)
    recon_audio = jnp.transpose(qa, (0, 2, 1))

    bi, di, h, w = image_input.shape
    xi = jnp.transpose(image_input, (0, 2, 3, 1)).reshape(bi, h * w, di)
    qi, image_indices, vq_image_loss = _rvq(xi, image_codebooks, 512)
    recon_image = jnp.transpose(qi.reshape(bi, h, w, di), (0, 3, 1, 2))

    return (recon_audio, recon_image, vq_audio_loss, vq_image_loss,
            audio_indices, image_indices)


# speedup vs baseline: 1.8709x; 1.0249x over previous
import jax
import jax.numpy as jnp
from jax.experimental import pallas as pl
from jax.experimental.pallas import tpu as pltpu

_K = 1024
_D = 64
_Q = 2


def _dot(a, b, dims):
    return jax.lax.dot_general(a, b, (dims, ((), ())),
                               preferred_element_type=jnp.float32,
                               precision=jax.lax.Precision.DEFAULT)


def _rvq_kernel(x_ref, cb_ref, sp_ref, cn_ref, rn_ref,
                out_ref, idx0_ref, idx1_ref, loss_ref):
    x = x_ref[0]
    residual = x
    quant_out = jnp.zeros_like(x)
    losses = []
    idx_refs = (idx0_ref, idx1_ref)
    for q in range(_Q):
        cb = cb_ref[q]
        cnorm = cn_ref[q]
        if q == 0:
            rnorm = rn_ref[0, 0]
        else:
            rnorm = jnp.sum(residual * residual, axis=1)
        prod = _dot(residual, cb, ((1,), (1,)))
        d = (rnorm[:, None] - 2.0 * prod) + cnorm[None, :]
        dmin = jnp.min(d, axis=1)
        iota = jax.lax.broadcasted_iota(jnp.int32, d.shape, 1)
        idx = jnp.min(jnp.where(d == dmin[:, None], iota, _K), axis=1)
        idx_refs[q][0, 0] = idx
        onehot = (iota == idx[:, None]).astype(jnp.float32)
        if q == 0:
            quant = ((_dot(onehot, sp_ref[0], ((1,), (0,)))
                      + _dot(onehot, sp_ref[1], ((1,), (0,))))
                     + _dot(onehot, sp_ref[2], ((1,), (0,))))
        else:
            quant = _dot(onehot, cb, ((1,), (0,)))
        diff = quant - residual
        losses.append(jnp.sum(diff * diff))
        quant_out = quant_out + (residual + (quant - residual))
        residual = residual - quant
    out_ref[0] = quant_out
    loss_ref[0] = jnp.stack([jnp.broadcast_to(l, (128,)) for l in losses])


def _split3(cb):
    mask = jnp.int32(-65536)
    hi = jnp.bitwise_and(cb.view(jnp.int32), mask).view(jnp.float32)
    r = cb - hi
    mid = jnp.bitwise_and(r.view(jnp.int32), mask).view(jnp.float32)
    lo = r - mid
    return jnp.stack([hi, mid, lo])


def _rvq(x, codebooks, tb, interpret=False):
    b, t, d_ = x.shape
    g = t // tb
    cnorm = jnp.sum(codebooks * codebooks, axis=-1)
    rnorm = jnp.sum(x * x, axis=-1).reshape(b * g, 1, tb)
    out, idx0, idx1, lossp = pl.pallas_call(
        _rvq_kernel,
        grid=(b, g),
        in_specs=[
            pl.BlockSpec((1, tb, d_), lambda i, j: (i, j, 0)),
            pl.BlockSpec((_Q, _K, d_), lambda i, j: (0, 0, 0)),
            pl.BlockSpec((3, _K, d_), lambda i, j: (0, 0, 0)),
            pl.BlockSpec((_Q, _K), lambda i, j: (0, 0)),
            pl.BlockSpec((1, 1, tb), lambda i, j: (i * g + j, 0, 0)),
        ],
        out_specs=[
            pl.BlockSpec((1, tb, d_), lambda i, j: (i, j, 0)),
            pl.BlockSpec((1, 1, tb), lambda i, j: (i * g + j, 0, 0)),
            pl.BlockSpec((1, 1, tb), lambda i, j: (i * g + j, 0, 0)),
            pl.BlockSpec((1, _Q, 128), lambda i, j: (i * g + j, 0, 0)),
        ],
        out_shape=[
            jax.ShapeDtypeStruct((b, t, d_), jnp.float32),
            jax.ShapeDtypeStruct((b * g, 1, tb), jnp.int32),
            jax.ShapeDtypeStruct((b * g, 1, tb), jnp.int32),
            jax.ShapeDtypeStruct((b * g, _Q, 128), jnp.float32),
        ],
        compiler_params=pltpu.CompilerParams(
            dimension_semantics=("parallel", "parallel")),
        interpret=interpret,
    )(x, codebooks, _split3(codebooks[0]), cnorm, rnorm)
    indices = jnp.stack([idx0.reshape(b, t), idx1.reshape(b, t)], axis=-1)
    loss = lossp[:, :, 0].sum(axis=0) / (b * t * d_)
    return out, indices, loss


def kernel(audio_input, image_input, audio_codebooks, image_codebooks):
    xa = jnp.transpose(audio_input, (0, 2, 1))
    qa, audio_indices, vq_audio_loss = _rvq(xa, audio_codebooks, 1024)
    recon_audio = jnp.transpose(qa, (0, 2, 1))

    bi, di, h, w = image_input.shape
    xi = jnp.transpose(image_input, (0, 2, 3, 1)).reshape(bi, h * w, di)
    qi, image_indices, vq_image_loss = _rvq(xi, image_codebooks, 1024)
    recon_image = jnp.transpose(qi.reshape(bi, h, w, di), (0, 3, 1, 2))

    return (recon_audio, recon_image, vq_audio_loss, vq_image_loss,
            audio_indices, image_indices)
